# Initial kernel scaffold; baseline (speedup 1.0000x reference)
import jax
import jax.numpy as jnp
from jax.experimental import pallas as pl


def kernel(words_as_ids, embedding_table):
    raise NotImplementedError("write your pallas kernel here")


# speedup vs baseline: 4.9451x; 4.9451x over previous
import functools

import jax
import jax.numpy as jnp
from jax import lax
from jax.experimental import pallas as pl
from jax.experimental.pallas import tpu as pltpu
from jax.experimental.pallas import tpu_sc as plsc

_NUM_CORES = 2
_NUM_SUBCORES = 16
_NUM_WORKERS = _NUM_CORES * _NUM_SUBCORES


@functools.partial(jax.jit, static_argnames=("chunk",))
def _embedding_lookup(table, flat_ids, chunk):
    total = flat_ids.shape[0]
    depth = table.shape[1]
    per_worker = total // _NUM_WORKERS
    n_chunks = per_worker // chunk

    mesh = plsc.VectorSubcoreMesh(
        core_axis_name="c",
        subcore_axis_name="s",
        num_cores=_NUM_CORES,
        num_subcores=_NUM_SUBCORES,
    )

    @functools.partial(
        pl.kernel,
        mesh=mesh,
        out_type=jax.ShapeDtypeStruct((total, depth), table.dtype),
        scratch_types=[
            pltpu.VMEM((chunk,), jnp.int32),
            pltpu.VMEM((chunk, depth), table.dtype),
            pltpu.SemaphoreType.DMA,
        ],
        compiler_params=pltpu.CompilerParams(use_tc_tiling_on_sc=False),
    )
    def emb_kernel(table_hbm, idx_hbm, out_hbm, idx_v, rows_v, sem):
        wid = lax.axis_index("s") * _NUM_CORES + lax.axis_index("c")
        base = wid * per_worker

        def body(i, carry):
            off = base + i * chunk
            pltpu.sync_copy(idx_hbm.at[pl.ds(off, chunk)], idx_v)
            pltpu.async_copy(table_hbm.at[idx_v], rows_v, sem).wait()
            pltpu.sync_copy(rows_v, out_hbm.at[pl.ds(off, chunk)])
            return carry

        lax.fori_loop(0, n_chunks, body, 0)

    return emb_kernel(table, flat_ids)


def kernel(words_as_ids, embedding_table):
    batch, hist = words_as_ids.shape
    flat_ids = words_as_ids.reshape(-1).astype(jnp.int32)
    out = _embedding_lookup(embedding_table, flat_ids, chunk=2048)
    return out.reshape(batch, hist, embedding_table.shape[1])


# speedup vs baseline: 4.9466x; 1.0003x over previous
import functools

import jax
import jax.numpy as jnp
from jax import lax
from jax.experimental import pallas as pl
from jax.experimental.pallas import tpu as pltpu
from jax.experimental.pallas import tpu_sc as plsc

_NUM_CORES = 2
_NUM_SUBCORES = 16
_NUM_WORKERS = _NUM_CORES * _NUM_SUBCORES

_NBUF = 4


@functools.partial(jax.jit, static_argnames=("chunk",))
def _embedding_lookup(table, flat_ids, chunk):
    total = flat_ids.shape[0]
    depth = table.shape[1]
    per_worker = total // _NUM_WORKERS
    n_chunks = per_worker // chunk
    n_groups = n_chunks // _NBUF

    mesh = plsc.VectorSubcoreMesh(
        core_axis_name="c",
        subcore_axis_name="s",
        num_cores=_NUM_CORES,
        num_subcores=_NUM_SUBCORES,
    )

    @functools.partial(
        pl.kernel,
        mesh=mesh,
        out_type=jax.ShapeDtypeStruct((total, depth), table.dtype),
        scratch_types=(
            [pltpu.VMEM((chunk,), jnp.int32) for _ in range(_NBUF)]
            + [pltpu.VMEM((chunk, depth), table.dtype) for _ in range(_NBUF)]
            + [pltpu.SemaphoreType.DMA((_NBUF,)),
               pltpu.SemaphoreType.DMA((_NBUF,))]
        ),
        compiler_params=pltpu.CompilerParams(use_tc_tiling_on_sc=False),
    )
    def emb_kernel(table_hbm, idx_hbm, out_hbm, *scratch):
        idx_v = scratch[:_NBUF]
        rows_v = scratch[_NBUF:2 * _NBUF]
        gsem, osem = scratch[2 * _NBUF], scratch[2 * _NBUF + 1]
        wid = lax.axis_index("s") * _NUM_CORES + lax.axis_index("c")
        base = wid * per_worker

        def start_chunk(j, b):
            off = base + j * chunk
            pltpu.sync_copy(idx_hbm.at[pl.ds(off, chunk)], idx_v[b])
            pltpu.async_copy(table_hbm.at[idx_v[b]], rows_v[b], gsem.at[b])

        def drain_chunk(j, b):
            pltpu.make_async_copy(table_hbm.at[idx_v[b]], rows_v[b],
                                  gsem.at[b]).wait()
            off = base + j * chunk
            pltpu.async_copy(rows_v[b], out_hbm.at[pl.ds(off, chunk)],
                             osem.at[b])

        def wait_out(j, b):
            off = base + j * chunk
            pltpu.make_async_copy(rows_v[b], out_hbm.at[pl.ds(off, chunk)],
                                  osem.at[b]).wait()

        for b in range(_NBUF):
            start_chunk(b, b)

        def body(g, carry):
            for b in range(_NBUF):
                j = g * _NBUF + b
                drain_chunk(j - _NBUF, b)
                wait_out(j - _NBUF, b)
                start_chunk(j, b)
            return carry

        lax.fori_loop(1, n_groups, body, 0)

        for b in range(_NBUF):
            j = (n_groups - 1) * _NBUF + b
            drain_chunk(j, b)
        for b in range(_NBUF):
            j = (n_groups - 1) * _NBUF + b
            wait_out(j, b)

    return emb_kernel(table, flat_ids)


def kernel(words_as_ids, embedding_table):
    batch, hist = words_as_ids.shape
    flat_ids = words_as_ids.reshape(-1).astype(jnp.int32)
    out = _embedding_lookup(embedding_table, flat_ids, chunk=800)
    return out.reshape(batch, hist, embedding_table.shape[1])
